# Initial kernel scaffold; baseline (speedup 1.0000x reference)
import jax
import jax.numpy as jnp
from jax.experimental import pallas as pl


def kernel(input_ids, token_type_ids, token_table, position_table, segment_table, gamma, beta):
    raise NotImplementedError("write your pallas kernel here")


# speedup vs baseline: 6.5310x; 6.5310x over previous
import functools

import jax
import jax.numpy as jnp
from jax import lax
from jax.experimental import pallas as pl
from jax.experimental.pallas import tpu as pltpu
from jax.experimental.pallas import tpu_sc as plsc

VOCAB = 100000
D = 128
SEQ = 200
BATCH = 1024
N = BATCH * SEQ
EPS = 1e-5

NC = 2
NS = 16
NW = NC * NS
B_PER_W = N // NW
CHUNK = 800


@functools.cache
def _make_sc_gather():
    mesh = plsc.VectorSubcoreMesh(core_axis_name="c", subcore_axis_name="s")

    @functools.partial(
        pl.kernel,
        mesh=mesh,
        out_type=jax.ShapeDtypeStruct((N, D), jnp.float32),
        scratch_types=[
            pltpu.VMEM((B_PER_W,), jnp.int32),
            pltpu.VMEM((CHUNK, D), jnp.float32),
            pltpu.SemaphoreType.DMA,
        ],
    )
    def gather_k(idx_hbm, table_hbm, out_hbm, idx_v, rows_v, sem):
        wid = lax.axis_index("s") * NC + lax.axis_index("c")
        base = wid * B_PER_W
        pltpu.sync_copy(idx_hbm.at[pl.ds(base, B_PER_W)], idx_v)
        for c in range(B_PER_W // CHUNK):
            pltpu.async_copy(
                table_hbm.at[idx_v.at[pl.ds(c * CHUNK, CHUNK)]], rows_v, sem
            ).wait()
            pltpu.sync_copy(rows_v, out_hbm.at[pl.ds(base + c * CHUNK, CHUNK)])

    return gather_k


def _ln_body(tok_ref, tt_ref, pos_ref, seg_ref, g_ref, b_ref, out_ref):
    tok = tok_ref[...]
    tt = tt_ref[...]
    pos = pos_ref[...]
    seg = seg_ref[...]
    segv = jnp.where((tt[..., None] == 0), seg[0][None, None, :], seg[1][None, None, :])
    emb = tok + pos[None, :, :] + segv
    mean = jnp.mean(emb, axis=-1, keepdims=True)
    cent = emb - mean
    var = jnp.mean(cent * cent, axis=-1, keepdims=True)
    out_ref[...] = cent * lax.rsqrt(var + EPS) * g_ref[...][None] + b_ref[...][None]


_BB = 8


def _tc_layernorm(tok, tt, pos, seg, gamma, beta):
    return pl.pallas_call(
        _ln_body,
        grid=(BATCH // _BB,),
        in_specs=[
            pl.BlockSpec((_BB, SEQ, D), lambda i: (i, 0, 0)),
            pl.BlockSpec((_BB, SEQ), lambda i: (i, 0)),
            pl.BlockSpec((SEQ, D), lambda i: (0, 0)),
            pl.BlockSpec((2, D), lambda i: (0, 0)),
            pl.BlockSpec((1, D), lambda i: (0, 0)),
            pl.BlockSpec((1, D), lambda i: (0, 0)),
        ],
        out_specs=pl.BlockSpec((_BB, SEQ, D), lambda i: (i, 0, 0)),
        out_shape=jax.ShapeDtypeStruct((BATCH, SEQ, D), jnp.float32),
    )(tok, tt, pos, seg, gamma, beta)


def kernel(input_ids, token_type_ids, token_table, position_table, segment_table, gamma, beta):
    ids_flat = input_ids.reshape(-1).astype(jnp.int32)
    tok = _make_sc_gather()(ids_flat, token_table).reshape(BATCH, SEQ, D)
    return _tc_layernorm(
        tok,
        token_type_ids.astype(jnp.int32),
        position_table[:SEQ],
        segment_table,
        gamma.reshape(1, D),
        beta.reshape(1, D),
    )


# speedup vs baseline: 7.6219x; 1.1670x over previous
import functools

import jax
import jax.numpy as jnp
from jax import lax
from jax.experimental import pallas as pl
from jax.experimental.pallas import tpu as pltpu
from jax.experimental.pallas import tpu_sc as plsc

VOCAB = 100000
D = 128
SEQ = 200
BATCH = 1024
N = BATCH * SEQ
EPS = 1e-5

NC = 2
NS = 16
NW = NC * NS
B_PER_W = N // NW
CHUNK = 800


@functools.cache
def _make_sc_gather():
    mesh = plsc.VectorSubcoreMesh(core_axis_name="c", subcore_axis_name="s")

    @functools.partial(
        pl.kernel,
        mesh=mesh,
        out_type=jax.ShapeDtypeStruct((N, D), jnp.float32),
        scratch_types=[
            pltpu.VMEM((B_PER_W,), jnp.int32),
            pltpu.VMEM((CHUNK, D), jnp.float32),
            pltpu.SemaphoreType.DMA,
        ],
    )
    def gather_k(idx_hbm, table_hbm, out_hbm, idx_v, rows_v, sem):
        wid = lax.axis_index("s") * NC + lax.axis_index("c")
        base = wid * B_PER_W
        pltpu.sync_copy(idx_hbm.at[pl.ds(base, B_PER_W)], idx_v)
        for c in range(B_PER_W // CHUNK):
            pltpu.async_copy(
                table_hbm.at[idx_v.at[pl.ds(c * CHUNK, CHUNK)]], rows_v, sem
            ).wait()
            pltpu.sync_copy(rows_v, out_hbm.at[pl.ds(base + c * CHUNK, CHUNK)])

    return gather_k


def _ln_body(tok_ref, tt_ref, pos_ref, seg_ref, g_ref, b_ref, out_ref):
    tok = tok_ref[...]
    tt = tt_ref[...]
    pos = pos_ref[...]
    seg = seg_ref[...]
    segv = jnp.where((tt[..., None] == 0), seg[0][None, None, :], seg[1][None, None, :])
    emb = tok + pos[None, :, :] + segv
    mean = jnp.mean(emb, axis=-1, keepdims=True)
    cent = emb - mean
    var = jnp.mean(cent * cent, axis=-1, keepdims=True)
    out_ref[...] = cent * lax.rsqrt(var + EPS) * g_ref[...][None] + b_ref[...][None]


_BB = 16


def _tc_layernorm(tok, tt, pos, seg, gamma, beta):
    return pl.pallas_call(
        _ln_body,
        grid=(BATCH // _BB,),
        in_specs=[
            pl.BlockSpec((_BB, SEQ, D), lambda i: (i, 0, 0)),
            pl.BlockSpec((_BB, SEQ), lambda i: (i, 0)),
            pl.BlockSpec((SEQ, D), lambda i: (0, 0)),
            pl.BlockSpec((2, D), lambda i: (0, 0)),
            pl.BlockSpec((1, D), lambda i: (0, 0)),
            pl.BlockSpec((1, D), lambda i: (0, 0)),
        ],
        out_specs=pl.BlockSpec((_BB, SEQ, D), lambda i: (i, 0, 0)),
        out_shape=jax.ShapeDtypeStruct((BATCH, SEQ, D), jnp.float32),
    )(tok, tt, pos, seg, gamma, beta)


def kernel(input_ids, token_type_ids, token_table, position_table, segment_table, gamma, beta):
    ids_flat = input_ids.reshape(-1).astype(jnp.int32)
    tok = _make_sc_gather()(ids_flat, token_table).reshape(BATCH, SEQ, D)
    return _tc_layernorm(
        tok,
        token_type_ids.astype(jnp.int32),
        position_table[:SEQ],
        segment_table,
        gamma.reshape(1, D),
        beta.reshape(1, D),
    )


# speedup vs baseline: 8.2775x; 1.0860x over previous
import functools

import jax
import jax.numpy as jnp
from jax import lax
from jax.experimental import pallas as pl
from jax.experimental.pallas import tpu as pltpu
from jax.experimental.pallas import tpu_sc as plsc

VOCAB = 100000
D = 128
SEQ = 200
BATCH = 1024
N = BATCH * SEQ
EPS = 1e-5

NC = 2
NS = 16
NW = NC * NS
B_PER_W = N // NW
CHUNK = 800


@functools.cache
def _make_sc_gather():
    mesh = plsc.VectorSubcoreMesh(core_axis_name="c", subcore_axis_name="s")

    @functools.partial(
        pl.kernel,
        mesh=mesh,
        out_type=jax.ShapeDtypeStruct((N, D), jnp.float32),
        scratch_types=[
            pltpu.VMEM((B_PER_W,), jnp.int32),
            pltpu.VMEM((CHUNK, D), jnp.float32),
            pltpu.SemaphoreType.DMA,
        ],
    )
    def gather_k(idx_hbm, table_hbm, out_hbm, idx_v, rows_v, sem):
        wid = lax.axis_index("s") * NC + lax.axis_index("c")
        base = wid * B_PER_W
        pltpu.sync_copy(idx_hbm.at[pl.ds(base, B_PER_W)], idx_v)
        for c in range(B_PER_W // CHUNK):
            pltpu.async_copy(
                table_hbm.at[idx_v.at[pl.ds(c * CHUNK, CHUNK)]], rows_v, sem
            ).wait()
            pltpu.sync_copy(rows_v, out_hbm.at[pl.ds(base + c * CHUNK, CHUNK)])

    return gather_k


def _ln_body(tok_ref, tt_ref, pos_ref, seg_ref, g_ref, b_ref, out_ref):
    tok = tok_ref[...]
    tt = tt_ref[...]
    pos = pos_ref[...]
    seg = seg_ref[...]
    segv = jnp.where((tt[..., None] == 0), seg[0][None, None, :], seg[1][None, None, :])
    emb = tok + pos[None, :, :] + segv
    mean = jnp.mean(emb, axis=-1, keepdims=True)
    cent = emb - mean
    var = jnp.mean(cent * cent, axis=-1, keepdims=True)
    out_ref[...] = cent * lax.rsqrt(var + EPS) * g_ref[...][None] + b_ref[...][None]


_BB = 32


def _tc_layernorm(tok, tt, pos, seg, gamma, beta):
    return pl.pallas_call(
        _ln_body,
        grid=(BATCH // _BB,),
        in_specs=[
            pl.BlockSpec((_BB, SEQ, D), lambda i: (i, 0, 0)),
            pl.BlockSpec((_BB, SEQ), lambda i: (i, 0)),
            pl.BlockSpec((SEQ, D), lambda i: (0, 0)),
            pl.BlockSpec((2, D), lambda i: (0, 0)),
            pl.BlockSpec((1, D), lambda i: (0, 0)),
            pl.BlockSpec((1, D), lambda i: (0, 0)),
        ],
        out_specs=pl.BlockSpec((_BB, SEQ, D), lambda i: (i, 0, 0)),
        out_shape=jax.ShapeDtypeStruct((BATCH, SEQ, D), jnp.float32),
    )(tok, tt, pos, seg, gamma, beta)


def kernel(input_ids, token_type_ids, token_table, position_table, segment_table, gamma, beta):
    ids_flat = input_ids.reshape(-1).astype(jnp.int32)
    tok = _make_sc_gather()(ids_flat, token_table).reshape(BATCH, SEQ, D)
    return _tc_layernorm(
        tok,
        token_type_ids.astype(jnp.int32),
        position_table[:SEQ],
        segment_table,
        gamma.reshape(1, D),
        beta.reshape(1, D),
    )


# speedup vs baseline: 8.5699x; 1.0353x over previous
import functools

import jax
import jax.numpy as jnp
from jax import lax
from jax.experimental import pallas as pl
from jax.experimental.pallas import tpu as pltpu
from jax.experimental.pallas import tpu_sc as plsc

VOCAB = 100000
D = 128
SEQ = 200
BATCH = 1024
N = BATCH * SEQ
EPS = 1e-5

NC = 2
NS = 16
NW = NC * NS
B_PER_W = N // NW
CHUNK = 800


@functools.cache
def _make_sc_gather():
    mesh = plsc.VectorSubcoreMesh(core_axis_name="c", subcore_axis_name="s")

    @functools.partial(
        pl.kernel,
        mesh=mesh,
        out_type=jax.ShapeDtypeStruct((N, D), jnp.float32),
        scratch_types=[
            pltpu.VMEM((B_PER_W,), jnp.int32),
            pltpu.VMEM((CHUNK, D), jnp.float32),
            pltpu.SemaphoreType.DMA,
        ],
    )
    def gather_k(idx_hbm, table_hbm, out_hbm, idx_v, rows_v, sem):
        wid = lax.axis_index("s") * NC + lax.axis_index("c")
        base = wid * B_PER_W
        pltpu.sync_copy(idx_hbm.at[pl.ds(base, B_PER_W)], idx_v)
        for c in range(B_PER_W // CHUNK):
            pltpu.async_copy(
                table_hbm.at[idx_v.at[pl.ds(c * CHUNK, CHUNK)]], rows_v, sem
            ).wait()
            pltpu.sync_copy(rows_v, out_hbm.at[pl.ds(base + c * CHUNK, CHUNK)])

    return gather_k


def _ln_body(tok_ref, tt_ref, pos_ref, seg_ref, g_ref, b_ref, out_ref):
    tok = tok_ref[...]
    tt = tt_ref[...]
    pos = pos_ref[...]
    seg = seg_ref[...]
    segv = jnp.where((tt[..., None] == 0), seg[0][None, None, :], seg[1][None, None, :])
    emb = tok + pos[None, :, :] + segv
    mean = jnp.mean(emb, axis=-1, keepdims=True)
    cent = emb - mean
    var = jnp.mean(cent * cent, axis=-1, keepdims=True)
    out_ref[...] = cent * lax.rsqrt(var + EPS) * g_ref[...][None] + b_ref[...][None]


_BB = 64


def _tc_layernorm(tok, tt, pos, seg, gamma, beta):
    return pl.pallas_call(
        _ln_body,
        grid=(BATCH // _BB,),
        in_specs=[
            pl.BlockSpec((_BB, SEQ, D), lambda i: (i, 0, 0)),
            pl.BlockSpec((_BB, SEQ), lambda i: (i, 0)),
            pl.BlockSpec((SEQ, D), lambda i: (0, 0)),
            pl.BlockSpec((2, D), lambda i: (0, 0)),
            pl.BlockSpec((1, D), lambda i: (0, 0)),
            pl.BlockSpec((1, D), lambda i: (0, 0)),
        ],
        out_specs=pl.BlockSpec((_BB, SEQ, D), lambda i: (i, 0, 0)),
        out_shape=jax.ShapeDtypeStruct((BATCH, SEQ, D), jnp.float32),
    )(tok, tt, pos, seg, gamma, beta)


def kernel(input_ids, token_type_ids, token_table, position_table, segment_table, gamma, beta):
    ids_flat = input_ids.reshape(-1).astype(jnp.int32)
    tok = _make_sc_gather()(ids_flat, token_table).reshape(BATCH, SEQ, D)
    return _tc_layernorm(
        tok,
        token_type_ids.astype(jnp.int32),
        position_table[:SEQ],
        segment_table,
        gamma.reshape(1, D),
        beta.reshape(1, D),
    )
